# baseline (speedup 1.0000x reference)
import functools

import jax
import jax.numpy as jnp
from jax import lax
from jax.experimental import pallas as pl
from jax.experimental.pallas import tpu as pltpu
from jax.experimental.pallas import tpu_sc as plsc


def kernel(idx, token_embedding_table, pos_encoding):
    B, T = idx.shape
    V, D = token_embedding_table.shape
    G = T // 2
    SEG_PER_SEQ = 2
    SEQ_PER_CHUNK = 2
    NSEG = SEG_PER_SEQ * SEQ_PER_CHUNK

    info = plsc.get_sparse_core_info()
    NC, NS = info.num_cores, info.num_subcores
    NW = NC * NS
    n_chunks = B // SEQ_PER_CHUNK
    chunks_per_w = n_chunks // NW
    segs_per_w = chunks_per_w * NSEG

    idx2 = idx.reshape(B * SEG_PER_SEQ, G)
    pe2d = pos_encoding.reshape(T, D)
    pe_idx = (jnp.arange(NSEG, dtype=jnp.int32)[:, None] % SEG_PER_SEQ) * G + \
        jnp.arange(G, dtype=jnp.int32)[None, :]

    mesh = plsc.VectorSubcoreMesh(core_axis_name="c", subcore_axis_name="s")

    @functools.partial(
        pl.kernel,
        mesh=mesh,
        compiler_params=pltpu.CompilerParams(use_tc_tiling_on_sc=False),
        out_type=jax.ShapeDtypeStruct((n_chunks, NSEG, G, D), jnp.float32),
        scratch_types=[
            pltpu.VMEM((segs_per_w, G), jnp.int32),
            pltpu.VMEM((NSEG, G), jnp.int32),
            pltpu.VMEM((2, NSEG, G, D), jnp.float32),
            pltpu.SemaphoreType.DMA,
            pltpu.SemaphoreType.DMA,
            pltpu.SemaphoreType.DMA,
            pltpu.SemaphoreType.DMA,
            pltpu.SemaphoreType.DMA,
            pltpu.SemaphoreType.DMA,
        ],
    )
    def run(idx_hbm, table_hbm, pe_hbm, pe_idx_hbm, out_hbm, idx_all, pe_ix,
            rows, g0, g1, o0, o1, p0, p1):
        wid = lax.axis_index("s") * NC + lax.axis_index("c")
        base_c = wid * chunks_per_w
        pltpu.sync_copy(pe_idx_hbm, pe_ix)
        pltpu.sync_copy(idx_hbm.at[pl.ds(wid * segs_per_w, segs_per_w)], idx_all)
        gsem = (g0, g1)
        osem = (o0, o1)
        psem = (p0, p1)

        def fire_gathers(t, s):
            for j in range(NSEG):
                pltpu.async_copy(
                    table_hbm.at[idx_all.at[t * NSEG + j]], rows.at[s].at[j],
                    gsem[s],
                )

        def wait_gathers(s):
            pltpu.make_async_copy(out_hbm.at[0], rows.at[s], gsem[s]).wait()

        def fire_pe(s):
            for j in range(NSEG):
                pltpu.async_copy(
                    pe_hbm.at[pe_ix.at[j]], rows.at[s].at[j], psem[s], add=True,
                )

        def wait_pe(s):
            pltpu.make_async_copy(out_hbm.at[0], rows.at[s], psem[s]).wait()

        def wait_out(s):
            pltpu.make_async_copy(out_hbm.at[0], rows.at[s], osem[s]).wait()

        def fire_out(t, s):
            pltpu.async_copy(rows.at[s], out_hbm.at[base_c + t], osem[s])

        def step(t, s, first):
            o = 1 - s
            if not first:
                wait_out(o)
            fire_gathers(t + 1, o)
            wait_gathers(s)
            fire_pe(s)
            wait_pe(s)
            fire_out(t, s)

        fire_gathers(0, 0)
        step(0, 0, first=True)

        def pair(p, carry):
            t1 = 2 * p + 1
            step(t1, 1, first=False)
            step(t1 + 1, 0, first=False)
            return carry

        lax.fori_loop(0, (chunks_per_w - 2) // 2, pair, 0)

        wait_gathers(1)
        fire_pe(1)
        wait_pe(1)
        fire_out(chunks_per_w - 1, 1)
        wait_out(0)
        wait_out(1)

    out = run(idx2, token_embedding_table, pe2d, pe_idx)
    return out.reshape(B, T, D)
